# baseline (speedup 1.0000x reference)
import functools

import jax
import jax.numpy as jnp
from jax import lax
from jax.experimental import pallas as pl
from jax.experimental.pallas import tpu as pltpu
from jax.experimental.pallas import tpu_sc as plsc

K = 20
N = 10000
NP_ROWS = 10240
NP_COLS = 10240
RB = 256
D = 128
CW = 1024
NCH = NP_COLS // CW
E_PAD = NP_ROWS * K
NEG_SLOPE = 0.2
BIG_I = 2**30
INF = float("inf")


def _ha_kernel(x_ref, w_ref, asrc_ref, adst_ref, h_ref, as_ref, ad_ref):
    h = jnp.dot(x_ref[...], w_ref[...], preferred_element_type=jnp.float32)
    h_ref[...] = h
    as_ref[...] = jnp.sum(h * asrc_ref[...], axis=1, keepdims=True)
    ad_ref[...] = jnp.sum(h * adst_ref[...], axis=1, keepdims=True)


def _run_a(xs_pad, W, att_src, att_dst):
    return pl.pallas_call(
        _ha_kernel,
        grid=(NP_ROWS // RB,),
        in_specs=[
            pl.BlockSpec((RB, D), lambda i: (i, 0)),
            pl.BlockSpec((D, D), lambda i: (0, 0)),
            pl.BlockSpec((1, D), lambda i: (0, 0)),
            pl.BlockSpec((1, D), lambda i: (0, 0)),
        ],
        out_specs=[
            pl.BlockSpec((RB, D), lambda i: (i, 0)),
            pl.BlockSpec((RB, 1), lambda i: (i, 0)),
            pl.BlockSpec((RB, 1), lambda i: (i, 0)),
        ],
        out_shape=[
            jax.ShapeDtypeStruct((NP_ROWS, D), jnp.float32),
            jax.ShapeDtypeStruct((NP_ROWS, 1), jnp.float32),
            jax.ShapeDtypeStruct((NP_ROWS, 1), jnp.float32),
        ],
    )(xs_pad, W, att_src, att_dst)


def _knn_kernel(pos_ref, post_ref, ad_ref, as_ref, h_ref,
                idx_ref, w_ref, msgs_ref, d_scr, ad_scr):
    blk = pl.program_id(0)
    pb = pos_ref[...]
    sq_r = jnp.sum(pb * pb, axis=1, keepdims=True)
    rowg = lax.broadcasted_iota(jnp.int32, (RB, 1), 0) + blk * RB
    valid = rowg < N

    def _build(c, carry):
        c0 = pl.multiple_of(c * CW, CW)
        ptc = post_ref[:, pl.ds(c0, CW)]
        dmm = jnp.dot(pb, ptc, preferred_element_type=jnp.float32)
        sq_c = jnp.sum(ptc * ptc, axis=0, keepdims=True)
        dd = sq_r + sq_c - 2.0 * dmm
        colc = lax.broadcasted_iota(jnp.int32, (RB, CW), 1) + c * CW
        dd = jnp.where((colc >= N) | (colc == rowg), INF, dd)
        d_scr[:, pl.ds(c0, CW)] = dd
        return carry
    lax.fori_loop(0, NCH, _build, 0)

    def _sel(t, carry):
        i_prev = carry

        def _scan(c, acc):
            m, i, ad = acc
            c0 = pl.multiple_of(c * CW, CW)
            dd = d_scr[:, pl.ds(c0, CW)]
            colc = lax.broadcasted_iota(jnp.int32, (RB, CW), 1) + c * CW
            dd = jnp.where(colc == i_prev, INF, dd)
            d_scr[:, pl.ds(c0, CW)] = dd
            vmin = jnp.min(dd, axis=1, keepdims=True)
            eq = dd == vmin
            ic = jnp.min(jnp.where(eq, colc, BIG_I), axis=1, keepdims=True)
            adw = jnp.min(jnp.where(eq, ad_ref[:, pl.ds(c0, CW)], INF),
                          axis=1, keepdims=True)
            take = (vmin < m) | ((vmin == m) & (ic < i))
            return (jnp.where(take, vmin, m),
                    jnp.where(take, ic, i),
                    jnp.where(take, adw, ad))

        m0 = jnp.full((RB, 1), INF, jnp.float32)
        i0 = jnp.full((RB, 1), BIG_I, jnp.int32)
        ad0 = jnp.zeros((RB, 1), jnp.float32)
        _, i_t, ad_t = lax.fori_loop(0, NCH, _scan, (m0, i0, ad0))
        for tt in range(K):
            @pl.when(t == tt)
            def _store():
                idx_ref[:, tt:tt + 1] = jnp.where(valid, i_t, 0)
                ad_scr[:, tt:tt + 1] = ad_t
        return i_t
    lax.fori_loop(0, K, _sel, jnp.full((RB, 1), -1, jnp.int32))

    ads = ad_scr[:, :K]
    alpha = as_ref[...] + ads
    alpha = jnp.where(alpha >= 0, alpha, NEG_SLOPE * alpha)
    w = jnp.where(valid, jnp.exp(alpha), 0.0)
    w_ref[...] = w
    h = h_ref[...]
    for j in range(K):
        msgs_ref[j] = w[:, j:j + 1] * h


def _run_b(pos_pad, post_pad, ad_row, as_col, h):
    return pl.pallas_call(
        _knn_kernel,
        grid=(NP_ROWS // RB,),
        in_specs=[
            pl.BlockSpec((RB, 8), lambda i: (i, 0)),
            pl.BlockSpec((8, NP_COLS), lambda i: (0, 0)),
            pl.BlockSpec((1, NP_COLS), lambda i: (0, 0)),
            pl.BlockSpec((RB, 1), lambda i: (i, 0)),
            pl.BlockSpec((RB, D), lambda i: (i, 0)),
        ],
        out_specs=[
            pl.BlockSpec((RB, K), lambda i: (i, 0)),
            pl.BlockSpec((RB, K), lambda i: (i, 0)),
            pl.BlockSpec((K, RB, D), lambda i: (0, i, 0)),
        ],
        out_shape=[
            jax.ShapeDtypeStruct((NP_ROWS, K), jnp.int32),
            jax.ShapeDtypeStruct((NP_ROWS, K), jnp.float32),
            jax.ShapeDtypeStruct((K, NP_ROWS, D), jnp.float32),
        ],
        scratch_shapes=[
            pltpu.VMEM((RB, NP_COLS), jnp.float32),
            pltpu.VMEM((RB, 128), jnp.float32),
        ],
    )(pos_pad, post_pad, ad_row, as_col, h)


NTILE = 32
EDGES_PER_TILE = E_PAD // NTILE
CHUNK = 128
CHUNKS_PER_TILE = EDGES_PER_TILE // CHUNK
ACC_STRIPE = NP_ROWS // 16


def _sc_scatter_kernel(msgs_hbm, idx_hbm, w_hbm, out_hbm, dpart_hbm,
                       stage_v, idx_v, idx_slab, w_slab, denom_v, acc_sh):
    c = lax.axis_index("c")
    s = lax.axis_index("s")
    tile = c * 16 + s

    def _zrow(r, _):
        def _zcol(v, _):
            stage_v[r, pl.ds(v * 16, 16)] = jnp.zeros((16,), jnp.float32)
            return ()
        return lax.fori_loop(0, D // 16, _zcol, ())
    lax.fori_loop(0, CHUNK, _zrow, ())

    def _zden(i, _):
        denom_v[pl.ds(i * 16, 16)] = jnp.zeros((16,), jnp.float32)
        return ()
    lax.fori_loop(0, NP_ROWS // 16, _zden, ())

    def _zstripe(z, _):
        pltpu.sync_copy(stage_v,
                        acc_sh.at[pl.ds(s * ACC_STRIPE + z * CHUNK, CHUNK)])
        return ()
    lax.fori_loop(0, ACC_STRIPE // CHUNK, _zstripe, ())
    plsc.subcore_barrier()

    base = tile * EDGES_PER_TILE
    pltpu.sync_copy(idx_hbm.at[pl.ds(base, EDGES_PER_TILE)], idx_slab)
    pltpu.sync_copy(w_hbm.at[pl.ds(base, EDGES_PER_TILE)], w_slab)

    def _den(i, _):
        iv = idx_slab[pl.ds(i * 16, 16)]
        wv = w_slab[pl.ds(i * 16, 16)]
        plsc.addupdate_scatter(denom_v, [iv], wv)
        return ()
    lax.fori_loop(0, EDGES_PER_TILE // 16, _den, ())

    def _chunk(i, _):
        off = base + i * CHUNK
        pltpu.sync_copy(idx_hbm.at[pl.ds(off, CHUNK)], idx_v)
        pltpu.sync_copy(msgs_hbm.at[pl.ds(off, CHUNK)], stage_v)
        pltpu.sync_copy(stage_v, acc_sh.at[idx_v], add=True)
        return ()
    lax.fori_loop(0, CHUNKS_PER_TILE, _chunk, ())
    plsc.subcore_barrier()

    pltpu.sync_copy(acc_sh.at[pl.ds(s * ACC_STRIPE, ACC_STRIPE)],
                    out_hbm.at[c, pl.ds(s * ACC_STRIPE, ACC_STRIPE)])
    pltpu.sync_copy(denom_v, dpart_hbm.at[tile])


def _run_c(msgs_flat, idx_flat, w_flat):
    mesh = plsc.VectorSubcoreMesh(core_axis_name="c", subcore_axis_name="s")
    f = functools.partial(
        pl.kernel,
        out_type=[
            jax.ShapeDtypeStruct((2, NP_ROWS, D), jnp.float32),
            jax.ShapeDtypeStruct((NTILE, NP_ROWS), jnp.float32),
        ],
        mesh=mesh,
        scratch_types=[
            pltpu.VMEM((CHUNK, D), jnp.float32),
            pltpu.VMEM((CHUNK,), jnp.int32),
            pltpu.VMEM((EDGES_PER_TILE,), jnp.int32),
            pltpu.VMEM((EDGES_PER_TILE,), jnp.float32),
            pltpu.VMEM((NP_ROWS,), jnp.float32),
            pltpu.VMEM_SHARED((NP_ROWS, D), jnp.float32),
        ],
        compiler_params=pltpu.CompilerParams(needs_layout_passes=False),
    )(_sc_scatter_kernel)
    return f(msgs_flat, idx_flat, w_flat)


DB = 512


def _fin_kernel(p_ref, dp_ref, bias_ref, out_ref):
    p = p_ref[0] + p_ref[1]
    dp = dp_ref[...]
    ones = jnp.ones((NTILE, 1), jnp.float32)
    dsum = lax.dot_general(dp, ones, (((0,), (0,)), ((), ())),
                           preferred_element_type=jnp.float32)
    out_ref[...] = p / (dsum + jnp.float32(1e-16)) + bias_ref[...]


def _run_d(partials, dpart, bias_row):
    return pl.pallas_call(
        _fin_kernel,
        grid=(NP_ROWS // DB,),
        in_specs=[
            pl.BlockSpec((2, DB, D), lambda i: (0, i, 0)),
            pl.BlockSpec((NTILE, DB), lambda i: (0, i)),
            pl.BlockSpec((1, D), lambda i: (0, 0)),
        ],
        out_specs=pl.BlockSpec((DB, D), lambda i: (i, 0)),
        out_shape=jax.ShapeDtypeStruct((NP_ROWS, D), jnp.float32),
    )(partials, dpart, bias_row)


def kernel(x, position, W, att_src, att_dst, bias):
    xs = jnp.squeeze(x, axis=0)
    xs_pad = jnp.pad(xs, ((0, NP_ROWS - N), (0, 0)))
    pos_pad = jnp.pad(position, ((0, NP_ROWS - N), (0, 5)))
    post_pad = jnp.pad(position.T, ((0, 5), (0, NP_COLS - N)))
    asr = att_src.reshape(1, D)
    adr = att_dst.reshape(1, D)

    h, as_col, ad_col = _run_a(xs_pad, W, asr, adr)
    ad_row = ad_col.reshape(1, NP_ROWS)

    idx, w, msgs = _run_b(pos_pad, post_pad, ad_row, as_col, h)
    msgs_flat = msgs.reshape(E_PAD, D)
    idx_flat = idx.T.reshape(E_PAD)
    w_flat = w.T.reshape(E_PAD)

    partials, dpart = _run_c(msgs_flat, idx_flat, w_flat)
    out = _run_d(partials, dpart, bias.reshape(1, D))
    return out[:N][None, ...]


# speedup vs baseline: 1.1357x; 1.1357x over previous
import functools

import jax
import jax.numpy as jnp
from jax import lax
from jax.experimental import pallas as pl
from jax.experimental.pallas import tpu as pltpu
from jax.experimental.pallas import tpu_sc as plsc

K = 20
N = 10000
NP_ROWS = 10240
NP_COLS = 10240
RB = 256
D = 128
CW = 1024
NCH = NP_COLS // CW
E_PAD = NP_ROWS * K
NEG_SLOPE = 0.2
BIG_I = 2**30
INF = float("inf")


def _ha_kernel(x_ref, w_ref, asrc_ref, adst_ref, h_ref, as_ref, ad_ref):
    h = jnp.dot(x_ref[...], w_ref[...], preferred_element_type=jnp.float32)
    h_ref[...] = h
    as_ref[...] = jnp.sum(h * asrc_ref[...], axis=1, keepdims=True)
    ad_ref[...] = jnp.sum(h * adst_ref[...], axis=1, keepdims=True)


def _run_a(xs_pad, W, att_src, att_dst):
    return pl.pallas_call(
        _ha_kernel,
        grid=(NP_ROWS // RB,),
        in_specs=[
            pl.BlockSpec((RB, D), lambda i: (i, 0)),
            pl.BlockSpec((D, D), lambda i: (0, 0)),
            pl.BlockSpec((1, D), lambda i: (0, 0)),
            pl.BlockSpec((1, D), lambda i: (0, 0)),
        ],
        out_specs=[
            pl.BlockSpec((RB, D), lambda i: (i, 0)),
            pl.BlockSpec((RB, 1), lambda i: (i, 0)),
            pl.BlockSpec((RB, 1), lambda i: (i, 0)),
        ],
        out_shape=[
            jax.ShapeDtypeStruct((NP_ROWS, D), jnp.float32),
            jax.ShapeDtypeStruct((NP_ROWS, 1), jnp.float32),
            jax.ShapeDtypeStruct((NP_ROWS, 1), jnp.float32),
        ],
    )(xs_pad, W, att_src, att_dst)


def _knn_kernel(pos_ref, post_ref, ad_ref, as_ref, h_ref,
                idx_ref, w_ref, msgs_ref, d_scr, ad_scr):
    blk = pl.program_id(0)
    pb = pos_ref[...]
    sq_r = jnp.sum(pb * pb, axis=1, keepdims=True)
    rowg = lax.broadcasted_iota(jnp.int32, (RB, 1), 0) + blk * RB
    valid = rowg < N

    def _build(c, carry):
        c0 = pl.multiple_of(c * CW, CW)
        ptc = post_ref[:, pl.ds(c0, CW)]
        dmm = jnp.dot(pb, ptc, preferred_element_type=jnp.float32)
        sq_c = jnp.sum(ptc * ptc, axis=0, keepdims=True)
        dd = sq_r + sq_c - 2.0 * dmm
        colc = lax.broadcasted_iota(jnp.int32, (RB, CW), 1) + c * CW
        dd = jnp.where((colc >= N) | (colc == rowg), INF, dd)
        d_scr[:, pl.ds(c0, CW)] = dd
        return carry
    lax.fori_loop(0, NCH, _build, 0)

    def _sel(t, carry):
        i_prev = carry

        def _scan(c, acc):
            m, p = acc
            c0 = pl.multiple_of(c * CW, CW)
            dd = d_scr[:, pl.ds(c0, CW)]
            colc = lax.broadcasted_iota(jnp.int32, (RB, CW), 1) + c * CW
            dd = jnp.where(colc == i_prev, INF, dd)
            d_scr[:, pl.ds(c0, CW)] = dd
            vmin = jnp.min(dd, axis=1, keepdims=True)
            eq = dd == vmin
            adc = ad_ref[:, pl.ds(c0, CW)]
            adq = ((jnp.clip(adc, -15.9, 15.9) + 16.0)
                   * 4096.0).astype(jnp.int32)
            packed_row = (adq << 14) | colc
            pw = jnp.min(jnp.where(eq, packed_row, 2**31 - 1), axis=1,
                         keepdims=True)
            take = (vmin < m) | ((vmin == m) & (pw < p))
            return (jnp.where(take, vmin, m), jnp.where(take, pw, p))

        m0 = jnp.full((RB, 1), INF, jnp.float32)
        p0 = jnp.full((RB, 1), 2**31 - 1, jnp.int32)
        _, p_t = lax.fori_loop(0, NCH, _scan, (m0, p0))
        i_t = p_t & 16383
        ad_t = (p_t >> 14).astype(jnp.float32) / 4096.0 - 16.0
        for tt in range(K):
            @pl.when(t == tt)
            def _store():
                idx_ref[:, tt:tt + 1] = jnp.where(valid, i_t, 0)
                ad_scr[:, tt:tt + 1] = ad_t
        return i_t
    lax.fori_loop(0, K, _sel, jnp.full((RB, 1), -1, jnp.int32))

    ads = ad_scr[:, :K]
    alpha = as_ref[...] + ads
    alpha = jnp.where(alpha >= 0, alpha, NEG_SLOPE * alpha)
    w = jnp.where(valid, jnp.exp(alpha), 0.0)
    w_ref[...] = w
    h = h_ref[...]
    for j in range(K):
        msgs_ref[j] = w[:, j:j + 1] * h


def _run_b(pos_pad, post_pad, ad_row, as_col, h):
    return pl.pallas_call(
        _knn_kernel,
        grid=(NP_ROWS // RB,),
        in_specs=[
            pl.BlockSpec((RB, 8), lambda i: (i, 0)),
            pl.BlockSpec((8, NP_COLS), lambda i: (0, 0)),
            pl.BlockSpec((1, NP_COLS), lambda i: (0, 0)),
            pl.BlockSpec((RB, 1), lambda i: (i, 0)),
            pl.BlockSpec((RB, D), lambda i: (i, 0)),
        ],
        out_specs=[
            pl.BlockSpec((RB, K), lambda i: (i, 0)),
            pl.BlockSpec((RB, K), lambda i: (i, 0)),
            pl.BlockSpec((K, RB, D), lambda i: (0, i, 0)),
        ],
        out_shape=[
            jax.ShapeDtypeStruct((NP_ROWS, K), jnp.int32),
            jax.ShapeDtypeStruct((NP_ROWS, K), jnp.float32),
            jax.ShapeDtypeStruct((K, NP_ROWS, D), jnp.float32),
        ],
        scratch_shapes=[
            pltpu.VMEM((RB, NP_COLS), jnp.float32),
            pltpu.VMEM((RB, 128), jnp.float32),
        ],
    )(pos_pad, post_pad, ad_row, as_col, h)


NTILE = 32
EDGES_PER_TILE = E_PAD // NTILE
CHUNK = 128
CHUNKS_PER_TILE = EDGES_PER_TILE // CHUNK
ACC_STRIPE = NP_ROWS // 16


def _sc_scatter_kernel(msgs_hbm, idx_hbm, w_hbm, out_hbm, dpart_hbm,
                       stage_v, idx_v, idx_slab, w_slab, denom_v, acc_sh):
    c = lax.axis_index("c")
    s = lax.axis_index("s")
    tile = c * 16 + s

    def _zrow(r, _):
        def _zcol(v, _):
            stage_v[r, pl.ds(v * 16, 16)] = jnp.zeros((16,), jnp.float32)
            return ()
        return lax.fori_loop(0, D // 16, _zcol, ())
    lax.fori_loop(0, CHUNK, _zrow, ())

    def _zden(i, _):
        denom_v[pl.ds(i * 16, 16)] = jnp.zeros((16,), jnp.float32)
        return ()
    lax.fori_loop(0, NP_ROWS // 16, _zden, ())

    def _zstripe(z, _):
        pltpu.sync_copy(stage_v,
                        acc_sh.at[pl.ds(s * ACC_STRIPE + z * CHUNK, CHUNK)])
        return ()
    lax.fori_loop(0, ACC_STRIPE // CHUNK, _zstripe, ())
    plsc.subcore_barrier()

    base = tile * EDGES_PER_TILE
    pltpu.sync_copy(idx_hbm.at[pl.ds(base, EDGES_PER_TILE)], idx_slab)
    pltpu.sync_copy(w_hbm.at[pl.ds(base, EDGES_PER_TILE)], w_slab)

    def _den(i, _):
        iv = idx_slab[pl.ds(i * 16, 16)]
        wv = w_slab[pl.ds(i * 16, 16)]
        plsc.addupdate_scatter(denom_v, [iv], wv)
        return ()
    lax.fori_loop(0, EDGES_PER_TILE // 16, _den, ())

    def _chunk(i, _):
        off = base + i * CHUNK
        pltpu.sync_copy(idx_hbm.at[pl.ds(off, CHUNK)], idx_v)
        pltpu.sync_copy(msgs_hbm.at[pl.ds(off, CHUNK)], stage_v)
        pltpu.sync_copy(stage_v, acc_sh.at[idx_v], add=True)
        return ()
    lax.fori_loop(0, CHUNKS_PER_TILE, _chunk, ())
    plsc.subcore_barrier()

    pltpu.sync_copy(acc_sh.at[pl.ds(s * ACC_STRIPE, ACC_STRIPE)],
                    out_hbm.at[c, pl.ds(s * ACC_STRIPE, ACC_STRIPE)])
    pltpu.sync_copy(denom_v, dpart_hbm.at[tile])


def _run_c(msgs_flat, idx_flat, w_flat):
    mesh = plsc.VectorSubcoreMesh(core_axis_name="c", subcore_axis_name="s")
    f = functools.partial(
        pl.kernel,
        out_type=[
            jax.ShapeDtypeStruct((2, NP_ROWS, D), jnp.float32),
            jax.ShapeDtypeStruct((NTILE, NP_ROWS), jnp.float32),
        ],
        mesh=mesh,
        scratch_types=[
            pltpu.VMEM((CHUNK, D), jnp.float32),
            pltpu.VMEM((CHUNK,), jnp.int32),
            pltpu.VMEM((EDGES_PER_TILE,), jnp.int32),
            pltpu.VMEM((EDGES_PER_TILE,), jnp.float32),
            pltpu.VMEM((NP_ROWS,), jnp.float32),
            pltpu.VMEM_SHARED((NP_ROWS, D), jnp.float32),
        ],
        compiler_params=pltpu.CompilerParams(needs_layout_passes=False),
    )(_sc_scatter_kernel)
    return f(msgs_flat, idx_flat, w_flat)


DB = 512


def _fin_kernel(p_ref, dp_ref, bias_ref, out_ref):
    p = p_ref[0] + p_ref[1]
    dp = dp_ref[...]
    ones = jnp.ones((NTILE, 1), jnp.float32)
    dsum = lax.dot_general(dp, ones, (((0,), (0,)), ((), ())),
                           preferred_element_type=jnp.float32)
    out_ref[...] = p / (dsum + jnp.float32(1e-16)) + bias_ref[...]


def _run_d(partials, dpart, bias_row):
    return pl.pallas_call(
        _fin_kernel,
        grid=(NP_ROWS // DB,),
        in_specs=[
            pl.BlockSpec((2, DB, D), lambda i: (0, i, 0)),
            pl.BlockSpec((NTILE, DB), lambda i: (0, i)),
            pl.BlockSpec((1, D), lambda i: (0, 0)),
        ],
        out_specs=pl.BlockSpec((DB, D), lambda i: (i, 0)),
        out_shape=jax.ShapeDtypeStruct((NP_ROWS, D), jnp.float32),
    )(partials, dpart, bias_row)


def kernel(x, position, W, att_src, att_dst, bias):
    xs = jnp.squeeze(x, axis=0)
    xs_pad = jnp.pad(xs, ((0, NP_ROWS - N), (0, 0)))
    pos_pad = jnp.pad(position, ((0, NP_ROWS - N), (0, 5)))
    post_pad = jnp.pad(position.T, ((0, 5), (0, NP_COLS - N)))
    asr = att_src.reshape(1, D)
    adr = att_dst.reshape(1, D)

    h, as_col, ad_col = _run_a(xs_pad, W, asr, adr)
    ad_row = ad_col.reshape(1, NP_ROWS)

    idx, w, msgs = _run_b(pos_pad, post_pad, ad_row, as_col, h)
    msgs_flat = msgs.reshape(E_PAD, D)
    idx_flat = idx.T.reshape(E_PAD)
    w_flat = w.T.reshape(E_PAD)

    partials, dpart = _run_c(msgs_flat, idx_flat, w_flat)
    out = _run_d(partials, dpart, bias.reshape(1, D))
    return out[:N][None, ...]
